# baseline (speedup 1.0000x reference)
import functools

import jax
import jax.numpy as jnp
from jax import lax
from jax.experimental import pallas as pl
from jax.experimental.pallas import tpu as pltpu
from jax.experimental.pallas import tpu_sc as plsc

B = 16384
K = 64
L = 16
NC = 2
NS = 16
NW = NC * NS
BPW = B // NW
CHUNK = 128
NCHUNK = BPW // CHUNK
NROW = 1000000
NHALF = NROW // 2
RB = 10000


def _tc_relayout(tableU, tableI):
    def body(au, bu, ai, bi, ou, oi):
        ou[:, 0:K] = au[...]
        ou[:, K:128] = bu[...]
        oi[:, 0:K] = ai[...]
        oi[:, K:128] = bi[...]

    top = pl.BlockSpec((RB, K), lambda i: (i, 0))
    bot = pl.BlockSpec((RB, K), lambda i: (i + NHALF // RB, 0))
    out = pl.BlockSpec((RB, 128), lambda i: (i, 0))
    return pl.pallas_call(
        body,
        grid=(NHALF // RB,),
        in_specs=[top, bot, top, bot],
        out_specs=[out, out],
        out_shape=[jax.ShapeDtypeStruct((NHALF, 128), jnp.float32)] * 2,
    )(tableU, tableU, tableI, tableI)


def _sc_diffs(sampleU, sampleI, sampleJ, betaI, gammaU2, gammaI2):
    mesh = plsc.VectorSubcoreMesh(core_axis_name="c", subcore_axis_name="s")

    @functools.partial(
        pl.kernel,
        out_type=jax.ShapeDtypeStruct((B,), jnp.float32),
        mesh=mesh,
        compiler_params=pltpu.CompilerParams(
            use_tc_tiling_on_sc=True, needs_layout_passes=False),
        scratch_types=[
            pltpu.VMEM((CHUNK,), jnp.int32),
            pltpu.VMEM((CHUNK,), jnp.int32),
            pltpu.VMEM((CHUNK,), jnp.int32),
            pltpu.VMEM((CHUNK,), jnp.int32),
            pltpu.VMEM((CHUNK,), jnp.int32),
            pltpu.VMEM((CHUNK,), jnp.int32),
            pltpu.VMEM((CHUNK, 128), jnp.float32),
            pltpu.VMEM((CHUNK, 128), jnp.float32),
            pltpu.VMEM((CHUNK, 128), jnp.float32),
            pltpu.VMEM((CHUNK,), jnp.float32),
            pltpu.VMEM((CHUNK,), jnp.float32),
            pltpu.VMEM((CHUNK,), jnp.float32),
            pltpu.SemaphoreType.DMA,
        ],
    )
    def sc_kernel(sU, sI, sJ, bI_hbm, gU_hbm, gI_hbm, diff_hbm,
                  idxU, blkU, idxI, blkI, idxJ, blkJ,
                  gU, gI, gJ, bIv, bJv, dv, sem):
        wid = lax.axis_index("s") * NC + lax.axis_index("c")
        base = wid * BPW
        lanes = jnp.arange(L, dtype=jnp.int32)

        def chunk_body(ci, carry):
            cbase = base + ci * CHUNK
            pltpu.sync_copy(sU.at[pl.ds(cbase, CHUNK)], idxU)
            pltpu.sync_copy(sI.at[pl.ds(cbase, CHUNK)], idxI)
            pltpu.sync_copy(sJ.at[pl.ds(cbase, CHUNK)], idxJ)
            for g in range(CHUNK // L):
                sl = pl.ds(g * L, L)
                u = idxU[sl]
                blkU[sl] = jnp.where(u >= NHALF, u - NHALF, u)
                i = idxI[sl]
                blkI[sl] = jnp.where(i >= NHALF, i - NHALF, i)
                j = idxJ[sl]
                blkJ[sl] = jnp.where(j >= NHALF, j - NHALF, j)
            cp1 = pltpu.async_copy(gU_hbm.at[blkU], gU, sem)
            cp2 = pltpu.async_copy(gI_hbm.at[blkI], gI, sem)
            cp3 = pltpu.async_copy(gI_hbm.at[blkJ], gJ, sem)
            cp4 = pltpu.async_copy(bI_hbm.at[idxI], bIv, sem)
            cp5 = pltpu.async_copy(bI_hbm.at[idxJ], bJv, sem)
            cp1.wait()
            cp2.wait()
            cp3.wait()
            cp4.wait()
            cp5.wait()

            for g in range(CHUNK // L):
                sl = pl.ds(g * L, L)
                svec = jnp.full((L,), g * L, jnp.int32) + lanes
                zero = jnp.zeros((L,), jnp.int32)
                k64 = jnp.full((L,), K, jnp.int32)
                cu = jnp.where(idxU[sl] >= NHALF, k64, zero)
                ci_ = jnp.where(idxI[sl] >= NHALF, k64, zero)
                cj = jnp.where(idxJ[sl] >= NHALF, k64, zero)
                acc = bIv[sl] - bJv[sl]
                for k in range(K):
                    kv = lax.bitwise_and(lanes + k, K - 1)
                    gu = plsc.load_gather(gU, [svec, cu + kv])
                    gi = plsc.load_gather(gI, [svec, ci_ + kv])
                    gj = plsc.load_gather(gJ, [svec, cj + kv])
                    acc = acc + gu * (gi - gj)
                dv[sl] = acc

            pltpu.sync_copy(dv, diff_hbm.at[pl.ds(cbase, CHUNK)])
            return carry

        lax.fori_loop(0, NCHUNK, chunk_body, 0)

    return sc_kernel(sampleU, sampleI, sampleJ, betaI, gammaU2, gammaI2)


def _tc_loss(diffs):
    def body(d_ref, out_ref):
        loss = -jnp.mean(jnp.log(jax.nn.sigmoid(d_ref[...])))
        out_ref[...] = loss.reshape(1, 1)

    out = pl.pallas_call(
        body,
        out_shape=jax.ShapeDtypeStruct((1, 1), jnp.float32),
    )(diffs.reshape(B // 128, 128))
    return out[0, 0]


def kernel(sampleU, sampleI, sampleJ, betaI, gammaU, gammaI):
    gU2, gI2 = _tc_relayout(gammaU, gammaI)
    diffs = _sc_diffs(sampleU, sampleI, sampleJ, betaI, gU2, gI2)
    return _tc_loss(diffs)


# speedup vs baseline: 1.6195x; 1.6195x over previous
import functools

import jax
import jax.numpy as jnp
from jax import lax
from jax.experimental import pallas as pl
from jax.experimental.pallas import tpu as pltpu
from jax.experimental.pallas import tpu_sc as plsc

B = 16384
K = 64
L = 16
NC = 2
NS = 16
NW = NC * NS
BPW = B // NW
CHUNK = 64
NCHUNK = BPW // CHUNK


def _sc_diffs(sampleU, sampleI, sampleJ, betaI, gammaU, gammaI):
    mesh = plsc.VectorSubcoreMesh(core_axis_name="c", subcore_axis_name="s")

    @functools.partial(
        pl.kernel,
        out_type=jax.ShapeDtypeStruct((B,), jnp.float32),
        mesh=mesh,
        compiler_params=pltpu.CompilerParams(
            use_tc_tiling_on_sc=True, needs_layout_passes=False),
        scratch_types=[
            pltpu.VMEM((CHUNK,), jnp.int32),
            pltpu.VMEM((CHUNK,), jnp.int32),
            pltpu.VMEM((CHUNK,), jnp.int32),
            pltpu.VMEM((CHUNK, K), jnp.float32),
            pltpu.VMEM((CHUNK, K), jnp.float32),
            pltpu.VMEM((CHUNK, K), jnp.float32),
            pltpu.VMEM((CHUNK,), jnp.float32),
            pltpu.VMEM((CHUNK,), jnp.float32),
            pltpu.VMEM((CHUNK,), jnp.float32),
            pltpu.SemaphoreType.DMA,
            pltpu.SemaphoreType.DMA,
        ],
    )
    def sc_kernel(sU, sI, sJ, bI_hbm, gU_hbm, gI_hbm, diff_hbm,
                  idxU, idxI, idxJ, gU, gI, gJ, bIv, bJv, dv, sem, sem2):
        wid = lax.axis_index("s") * NC + lax.axis_index("c")
        base = wid * BPW
        lanes = jnp.arange(L, dtype=jnp.int32)

        def chunk_body(ci, carry):
            cbase = base + ci * CHUNK
            pltpu.sync_copy(sU.at[pl.ds(cbase, CHUNK)], idxU)
            pltpu.sync_copy(sI.at[pl.ds(cbase, CHUNK)], idxI)
            pltpu.sync_copy(sJ.at[pl.ds(cbase, CHUNK)], idxJ)
            cp4 = pltpu.async_copy(bI_hbm.at[idxI], bIv, sem2)
            cp5 = pltpu.async_copy(bI_hbm.at[idxJ], bJv, sem2)

            for g in range(CHUNK // L):
                sl = pl.ds(g * L, L)
                vu = idxU[sl]
                vi = idxI[sl]
                vj = idxJ[sl]
                for l in range(L):
                    s = g * L + l
                    pltpu.async_copy(gU_hbm.at[vu[l]], gU.at[s], sem)
                    pltpu.async_copy(gI_hbm.at[vi[l]], gI.at[s], sem)
                    pltpu.async_copy(gI_hbm.at[vj[l]], gJ.at[s], sem)
            for s in range(CHUNK):
                pltpu.make_async_copy(gU_hbm.at[0], gU.at[s], sem).wait()
                pltpu.make_async_copy(gU_hbm.at[0], gI.at[s], sem).wait()
                pltpu.make_async_copy(gU_hbm.at[0], gJ.at[s], sem).wait()
            cp4.wait()
            cp5.wait()

            for g in range(CHUNK // L):
                sl = pl.ds(g * L, L)
                svec = jnp.full((L,), g * L, jnp.int32) + lanes
                acc = bIv[sl] - bJv[sl]
                for k in range(K):
                    kv = lax.bitwise_and(lanes + k, K - 1)
                    gu = plsc.load_gather(gU, [svec, kv])
                    gi = plsc.load_gather(gI, [svec, kv])
                    gj = plsc.load_gather(gJ, [svec, kv])
                    acc = acc + gu * (gi - gj)
                dv[sl] = acc

            pltpu.sync_copy(dv, diff_hbm.at[pl.ds(cbase, CHUNK)])
            return carry

        lax.fori_loop(0, NCHUNK, chunk_body, 0)

    return sc_kernel(sampleU, sampleI, sampleJ, betaI, gammaU, gammaI)


def _tc_loss(diffs):
    def body(d_ref, out_ref):
        loss = -jnp.mean(jnp.log(jax.nn.sigmoid(d_ref[...])))
        out_ref[...] = loss.reshape(1, 1)

    out = pl.pallas_call(
        body,
        out_shape=jax.ShapeDtypeStruct((1, 1), jnp.float32),
    )(diffs.reshape(B // 128, 128))
    return out[0, 0]


def kernel(sampleU, sampleI, sampleJ, betaI, gammaU, gammaI):
    diffs = _sc_diffs(sampleU, sampleI, sampleJ, betaI, gammaU, gammaI)
    return _tc_loss(diffs)


# speedup vs baseline: 2.4034x; 1.4840x over previous
import functools

import jax
import jax.numpy as jnp
from jax import lax
from jax.experimental import pallas as pl
from jax.experimental.pallas import tpu as pltpu
from jax.experimental.pallas import tpu_sc as plsc

B = 16384
K = 64
L = 16
NC = 2
NS = 16
NW = NC * NS
BPW = B // NW
CHUNK = 64
NCHUNK = BPW // CHUNK


def _sc_diffs(sampleU, sampleI, sampleJ, betaI, gammaU, gammaI):
    mesh = plsc.VectorSubcoreMesh(core_axis_name="c", subcore_axis_name="s")

    @functools.partial(
        pl.kernel,
        out_type=jax.ShapeDtypeStruct((B,), jnp.float32),
        mesh=mesh,
        compiler_params=pltpu.CompilerParams(
            use_tc_tiling_on_sc=True, needs_layout_passes=False),
        scratch_types=[
            pltpu.VMEM((CHUNK,), jnp.int32),
            pltpu.VMEM((CHUNK,), jnp.int32),
            pltpu.VMEM((CHUNK,), jnp.int32),
            pltpu.VMEM((CHUNK, K), jnp.float32),
            pltpu.VMEM((CHUNK, K), jnp.float32),
            pltpu.VMEM((CHUNK, K), jnp.float32),
            pltpu.VMEM((CHUNK,), jnp.float32),
            pltpu.VMEM((CHUNK,), jnp.float32),
            pltpu.VMEM((CHUNK,), jnp.float32),
            pltpu.SemaphoreType.DMA,
            pltpu.SemaphoreType.DMA,
        ],
    )
    def sc_kernel(sU, sI, sJ, bI_hbm, gU_hbm, gI_hbm, diff_hbm,
                  idxU, idxI, idxJ, gU, gI, gJ, bIv, bJv, dv, sem, sem2):
        wid = lax.axis_index("s") * NC + lax.axis_index("c")
        base = wid * BPW
        lanes = jnp.arange(L, dtype=jnp.int32)

        def chunk_body(ci, carry):
            cbase = base + ci * CHUNK
            pltpu.sync_copy(sU.at[pl.ds(cbase, CHUNK)], idxU)
            pltpu.sync_copy(sI.at[pl.ds(cbase, CHUNK)], idxI)
            pltpu.sync_copy(sJ.at[pl.ds(cbase, CHUNK)], idxJ)
            cp4 = pltpu.async_copy(bI_hbm.at[idxI], bIv, sem2)
            cp5 = pltpu.async_copy(bI_hbm.at[idxJ], bJv, sem2)

            for g in range(CHUNK // L):
                sl = pl.ds(g * L, L)
                vu = idxU[sl]
                vi = idxI[sl]
                vj = idxJ[sl]
                for l in range(L):
                    s = g * L + l
                    u = vu[l]
                    i = vi[l]
                    j = vj[l]
                    pltpu.async_copy(
                        gU_hbm.at[lax.shift_right_logical(u, 3),
                                  lax.bitwise_and(u, 7)], gU.at[s], sem)
                    pltpu.async_copy(
                        gI_hbm.at[lax.shift_right_logical(i, 3),
                                  lax.bitwise_and(i, 7)], gI.at[s], sem)
                    pltpu.async_copy(
                        gI_hbm.at[lax.shift_right_logical(j, 3),
                                  lax.bitwise_and(j, 7)], gJ.at[s], sem)
            for s in range(CHUNK):
                pltpu.make_async_copy(gU_hbm.at[0, 0], gU.at[s], sem).wait()
                pltpu.make_async_copy(gU_hbm.at[0, 0], gI.at[s], sem).wait()
                pltpu.make_async_copy(gU_hbm.at[0, 0], gJ.at[s], sem).wait()
            cp4.wait()
            cp5.wait()

            for g in range(CHUNK // L):
                sl = pl.ds(g * L, L)
                svec = jnp.full((L,), g * L, jnp.int32) + lanes
                acc = bIv[sl] - bJv[sl]
                for k in range(K):
                    kv = lax.bitwise_and(lanes + k, K - 1)
                    gu = plsc.load_gather(gU, [svec, kv])
                    gi = plsc.load_gather(gI, [svec, kv])
                    gj = plsc.load_gather(gJ, [svec, kv])
                    acc = acc + gu * (gi - gj)
                dv[sl] = acc

            pltpu.sync_copy(dv, diff_hbm.at[pl.ds(cbase, CHUNK)])
            return carry

        lax.fori_loop(0, NCHUNK, chunk_body, 0)

    return sc_kernel(sampleU, sampleI, sampleJ, betaI, gammaU, gammaI)


def _tc_loss(diffs):
    def body(d_ref, out_ref):
        loss = -jnp.mean(jnp.log(jax.nn.sigmoid(d_ref[...])))
        out_ref[...] = loss.reshape(1, 1)

    out = pl.pallas_call(
        body,
        out_shape=jax.ShapeDtypeStruct((1, 1), jnp.float32),
    )(diffs.reshape(B // 128, 128))
    return out[0, 0]


def kernel(sampleU, sampleI, sampleJ, betaI, gammaU, gammaI):
    gU3 = gammaU.reshape(1000000 // 8, 8, K)
    gI3 = gammaI.reshape(1000000 // 8, 8, K)
    diffs = _sc_diffs(sampleU, sampleI, sampleJ, betaI, gU3, gI3)
    return _tc_loss(diffs)
